# Initial kernel scaffold; baseline (speedup 1.0000x reference)
import jax
import jax.numpy as jnp
from jax.experimental import pallas as pl


def kernel(x, edge_index, edge_weight, batch, global_features, W1, b1, g1, be1, m1, v1, W2, b2, g2, be2, m2, v2, Ws1, bs1, gs1, bes1, ms1, vs1, Ws2, bs2, gs2, bes2, ms2, vs2, Wo, bo, Wb, bb):
    raise NotImplementedError("write your pallas kernel here")


# speedup vs baseline: 16.1648x; 16.1648x over previous
import jax
import jax.numpy as jnp
from jax import lax
from jax.experimental import pallas as pl
from jax.experimental.pallas import tpu as pltpu
from jax.experimental.pallas import tpu_sc as plsc

_N = 100000
_E = 3200000
_B = 10000
_H = 32
_NC = 2
_NS = 16
_EP_ROWS = 25088
_EP = _EP_ROWS * 128
_N2 = 100096
_B2 = 10112
_NP = 102400
_NPG = _NP // 128

_mesh = plsc.VectorSubcoreMesh(core_axis_name="c", subcore_axis_name="s")
_f32 = jnp.float32
_SC_PARAMS = pltpu.CompilerParams(use_tc_tiling_on_sc=False)


def _deg_body(c_hbm, w_hbm, out_hbm, acc, zbuf, cstage, wstage):
    core = lax.axis_index("c")
    sub = lax.axis_index("s")
    wid = core * _NS + sub
    n_sl = _N2 // _NS
    zv = jnp.zeros((16,), _f32)

    @pl.loop(0, 64)
    def _(i):
        zbuf[pl.ds(i * 16, 16)] = zv

    @pl.loop(0, 6)
    def _(k):
        pltpu.sync_copy(zbuf, acc.at[pl.ds(sub * n_sl + k * 1024, 1024)])

    pltpu.sync_copy(zbuf.at[pl.ds(0, 112)],
                    acc.at[pl.ds(sub * n_sl + 6144, 112)])
    plsc.subcore_barrier()
    rows_per_tile = _EP_ROWS // (_NC * _NS)

    @pl.loop(0, rows_per_tile // 8)
    def _(g):
        row0 = wid * rows_per_tile + g * 8
        pltpu.sync_copy(c_hbm.at[pl.ds(row0, 8)], cstage)
        pltpu.sync_copy(w_hbm.at[pl.ds(row0, 8)], wstage)
        for j in range(8):
            pltpu.sync_copy(wstage.at[j], acc.at[cstage.at[j]], add=True)

    plsc.subcore_barrier()
    base = core * _N2 + sub * n_sl

    @pl.loop(0, 6)
    def _(k):
        pltpu.sync_copy(acc.at[pl.ds(sub * n_sl + k * 1024, 1024)], zbuf)
        pltpu.sync_copy(zbuf, out_hbm.at[pl.ds(base + k * 1024, 1024)])

    pltpu.sync_copy(acc.at[pl.ds(sub * n_sl + 6144, 112)],
                    zbuf.at[pl.ds(0, 112)])
    pltpu.sync_copy(zbuf.at[pl.ds(0, 112)],
                    out_hbm.at[pl.ds(base + 6144, 112)])


def _deg_call(c2, w2):
    return pl.kernel(
        _deg_body,
        out_type=jax.ShapeDtypeStruct((_NC * _N2,), _f32),
        mesh=_mesh,
        scratch_types=[
            pltpu.VMEM_SHARED((_N2,), _f32),
            pltpu.VMEM((1024,), _f32),
            pltpu.VMEM((8, 128), jnp.int32),
            pltpu.VMEM((8, 128), _f32),
        ],
        name="sc_deg",
        compiler_params=_SC_PARAMS,
    )(c2, w2)


def _conv_body(r_hbm, c_hbm, w_hbm, hp_hbm, out_hbm,
               acc, zbuf, rstage, cstage, wstage, rows):
    core = lax.axis_index("c")
    sub = lax.axis_index("s")
    n_sl = _N2 // _NS
    zv = jnp.zeros((16,), _f32)

    @pl.loop(0, 1024)
    def _(i):
        zbuf[pl.ds(i, 1), :] = zv.reshape(1, 16)

    @pl.loop(0, 6)
    def _(k):
        pltpu.sync_copy(zbuf, acc.at[pl.ds(sub * n_sl + k * 1024, 1024)])

    pltpu.sync_copy(zbuf.at[pl.ds(0, 112)],
                    acc.at[pl.ds(sub * n_sl + 6144, 112)])
    plsc.subcore_barrier()
    rows_per_tile = _EP_ROWS // _NS

    @pl.loop(0, rows_per_tile // 8)
    def _(g):
        row0 = sub * rows_per_tile + g * 8
        pltpu.sync_copy(r_hbm.at[pl.ds(row0, 8)], rstage)
        pltpu.sync_copy(c_hbm.at[pl.ds(row0, 8)], cstage)
        pltpu.sync_copy(w_hbm.at[pl.ds(row0, 8)], wstage)

        @pl.loop(0, 8)
        def _(j):
            pltpu.sync_copy(hp_hbm.at[core].at[rstage.at[j]], rows)

            @pl.loop(0, 8)
            def _(q):
                w16 = wstage.at[j][pl.ds(q * 16, 16)]
                for e in range(16):
                    idx = q * 16 + e
                    rows[pl.ds(idx, 1), :] = rows[pl.ds(idx, 1), :] * w16[e]

            pltpu.sync_copy(rows, acc.at[cstage.at[j]], add=True)

    plsc.subcore_barrier()

    @pl.loop(0, 6)
    def _(k):
        pltpu.sync_copy(acc.at[pl.ds(sub * n_sl + k * 1024, 1024)], zbuf)
        pltpu.sync_copy(zbuf, out_hbm.at[core, pl.ds(sub * n_sl + k * 1024,
                                                     1024)])

    pltpu.sync_copy(acc.at[pl.ds(sub * n_sl + 6144, 112)],
                    zbuf.at[pl.ds(0, 112)])
    pltpu.sync_copy(zbuf.at[pl.ds(0, 112)],
                    out_hbm.at[core, pl.ds(sub * n_sl + 6144, 112)])


def _conv_call(r2, c2, w2, hps):
    return pl.kernel(
        _conv_body,
        out_type=jax.ShapeDtypeStruct((_NC, _N2, 16), _f32),
        mesh=_mesh,
        scratch_types=[
            pltpu.VMEM_SHARED((_N2, 16), _f32),
            pltpu.VMEM((1024, 16), _f32),
            pltpu.VMEM((8, 128), jnp.int32),
            pltpu.VMEM((8, 128), jnp.int32),
            pltpu.VMEM((8, 128), _f32),
            pltpu.VMEM((128, 16), _f32),
        ],
        name="sc_conv",
        compiler_params=_SC_PARAMS,
    )(r2, c2, w2, hps)


_B2S = 10240


def _pool_body(z_hbm, b_hbm, v_hbm, sums_hbm, cnts_hbm,
               accS, accC, zbuf, zbufc, zstage, bstage, vstage):
    core = lax.axis_index("c")
    sub = lax.axis_index("s")
    wid = core * _NS + sub
    b_sl = _B2S // _NS
    b_sl2 = _B2 // _NS
    zv = jnp.zeros((16,), _f32)

    @pl.loop(0, 640)
    def _(i):
        zbuf[pl.ds(i, 1), :] = jnp.zeros((1, _H), _f32)

    @pl.loop(0, 40)
    def _(i):
        zbufc[pl.ds(i * 16, 16)] = zv

    pltpu.sync_copy(zbuf, accS.at[pl.ds(sub * b_sl, b_sl)])
    pltpu.sync_copy(zbufc.at[pl.ds(0, b_sl2)],
                    accC.at[pl.ds(sub * b_sl2, b_sl2)])
    plsc.subcore_barrier()
    groups_per_tile = _NPG // (_NC * _NS)

    @pl.loop(0, groups_per_tile)
    def _(g):
        grp = wid * groups_per_tile + g
        pltpu.sync_copy(z_hbm.at[pl.ds(grp * 128, 128)], zstage)
        pltpu.sync_copy(b_hbm.at[pl.ds(grp, 1)], bstage)
        pltpu.sync_copy(v_hbm.at[pl.ds(grp, 1)], vstage)
        pltpu.sync_copy(zstage, accS.at[bstage.at[0]], add=True)
        pltpu.sync_copy(vstage.at[0], accC.at[bstage.at[0]], add=True)

    plsc.subcore_barrier()
    pltpu.sync_copy(accS.at[pl.ds(sub * b_sl, b_sl)], zbuf)
    pltpu.sync_copy(zbuf, sums_hbm.at[pl.ds(core * _B2S + sub * b_sl, b_sl)])
    pltpu.sync_copy(accC.at[pl.ds(sub * b_sl2, b_sl2)],
                    zbufc.at[pl.ds(0, b_sl2)])
    pltpu.sync_copy(zbufc.at[pl.ds(0, b_sl2)],
                    cnts_hbm.at[pl.ds(core * _B2 + sub * b_sl2, b_sl2)])


def _pool_call(z2p, bp, vp):
    return pl.kernel(
        _pool_body,
        out_type=(jax.ShapeDtypeStruct((_NC * _B2S, _H), _f32),
                  jax.ShapeDtypeStruct((_NC * _B2,), _f32)),
        mesh=_mesh,
        scratch_types=[
            pltpu.VMEM_SHARED((_B2S, _H), _f32),
            pltpu.VMEM_SHARED((_B2,), _f32),
            pltpu.VMEM((640, _H), _f32),
            pltpu.VMEM((640,), _f32),
            pltpu.VMEM((128, _H), _f32),
            pltpu.VMEM((1, 128), jnp.int32),
            pltpu.VMEM((1, 128), _f32),
        ],
        name="sc_pool",
        compiler_params=_SC_PARAMS,
    )(z2p, bp, vp)


_BLK = 10000


def _tcb_body(dp_ref, x_ref, w1_ref, dis_ref, hp_ref):
    deg = 1.0 + dp_ref[:, 0] + dp_ref[:, 1]
    dis = lax.rsqrt(deg)
    h = jnp.dot(x_ref[...], w1_ref[...], preferred_element_type=_f32)
    hp_ref[...] = h * dis[:, None]
    dis_ref[...] = dis[:, None]


def _tcb_call(degp, x, W1):
    grid = (_N // _BLK,)
    return pl.pallas_call(
        _tcb_body,
        grid=grid,
        in_specs=[
            pl.BlockSpec((_BLK, _NC), lambda i: (i, 0)),
            pl.BlockSpec((_BLK, 13), lambda i: (i, 0)),
            pl.BlockSpec((13, _H), lambda i: (0, 0)),
        ],
        out_specs=[
            pl.BlockSpec((_BLK, 1), lambda i: (i, 0)),
            pl.BlockSpec((_BLK, _H), lambda i: (i, 0)),
        ],
        out_shape=[
            jax.ShapeDtypeStruct((_N, 1), _f32),
            jax.ShapeDtypeStruct((_N, _H), _f32),
        ],
        name="tc_prep",
    )(degp, x, W1)


def _tcmid_body(a_ref, hp_ref, dis_ref, b_ref, g_ref, be_ref, m_ref, v_ref,
                w2_ref, out_ref, *, matmul):
    agg = jnp.concatenate([a_ref[0], a_ref[1]], axis=-1)
    dis = dis_ref[...]
    conv = dis * (agg + hp_ref[...]) + b_ref[...]
    z = jnp.maximum(
        (conv - m_ref[...]) * lax.rsqrt(v_ref[...] + 1e-5) * g_ref[...]
        + be_ref[...], 0.0)
    if matmul:
        h2 = jnp.dot(z, w2_ref[...], preferred_element_type=_f32)
        out_ref[...] = h2 * dis
    else:
        out_ref[...] = z


def _tcmid_call(agg, hp, dis2d, b, g, be, m, v, W2, matmul, name):
    import functools
    grid = (_N // _BLK,)
    body = functools.partial(_tcmid_body, matmul=matmul)
    return pl.pallas_call(
        body,
        grid=grid,
        in_specs=[
            pl.BlockSpec((_NC, _BLK, 16), lambda i: (0, i, 0)),
            pl.BlockSpec((_BLK, _H), lambda i: (i, 0)),
            pl.BlockSpec((_BLK, 1), lambda i: (i, 0)),
            pl.BlockSpec((1, _H), lambda i: (0, 0)),
            pl.BlockSpec((1, _H), lambda i: (0, 0)),
            pl.BlockSpec((1, _H), lambda i: (0, 0)),
            pl.BlockSpec((1, _H), lambda i: (0, 0)),
            pl.BlockSpec((1, _H), lambda i: (0, 0)),
            pl.BlockSpec((_H, _H), lambda i: (0, 0)),
        ],
        out_specs=pl.BlockSpec((_BLK, _H), lambda i: (i, 0)),
        out_shape=jax.ShapeDtypeStruct((_N, _H), _f32),
        name=name,
    )(agg, hp, dis2d, b, g, be, m, v, W2)


def _tch_body(sums_ref, cnts_ref, gf_ref,
              ws1_ref, bs1_ref, gs1_ref, bes1_ref, ms1_ref, vs1_ref,
              ws2_ref, bs2_ref, gs2_ref, bes2_ref, ms2_ref, vs2_ref,
              wo_ref, bo_ref, wb_ref, bb_ref, or_ref, bl_ref):
    sums = sums_ref[0] + sums_ref[1]
    cnts = cnts_ref[0] + cnts_ref[1]
    ge = sums / jnp.clip(cnts, 1.0)
    comb = jnp.concatenate([ge, gf_ref[...]], axis=1)
    s = jnp.dot(comb, ws1_ref[...], preferred_element_type=_f32) + bs1_ref[...]
    s = jnp.maximum(
        (s - ms1_ref[...]) * lax.rsqrt(vs1_ref[...] + 1e-5) * gs1_ref[...]
        + bes1_ref[...], 0.0)
    s = jnp.dot(s, ws2_ref[...], preferred_element_type=_f32) + bs2_ref[...]
    s = jnp.maximum(
        (s - ms2_ref[...]) * lax.rsqrt(vs2_ref[...] + 1e-5) * gs2_ref[...]
        + bes2_ref[...], 0.0)
    or_ref[...] = jax.nn.sigmoid(
        jnp.dot(s, wo_ref[...], preferred_element_type=_f32) + bo_ref[...])
    bl_ref[...] = jax.nn.sigmoid(
        jnp.dot(s, wb_ref[...], preferred_element_type=_f32) + bb_ref[...])


def _tch_call(sums, cnts3, gf, Ws1, bs1, gs1, bes1, ms1, vs1,
              Ws2, bs2, gs2, bes2, ms2, vs2, Wo, bo, Wb, bb):
    return pl.pallas_call(
        _tch_body,
        out_shape=[
            jax.ShapeDtypeStruct((_B, 1), _f32),
            jax.ShapeDtypeStruct((_B, 1), _f32),
        ],
        name="tc_head",
    )(sums, cnts3, gf, Ws1, bs1, gs1, bes1, ms1, vs1,
      Ws2, bs2, gs2, bes2, ms2, vs2, Wo, bo, Wb, bb)


def kernel(x, edge_index, edge_weight, batch, global_features,
           W1, b1, g1, be1, m1, v1,
           W2, b2, g2, be2, m2, v2,
           Ws1, bs1, gs1, bes1, ms1, vs1,
           Ws2, bs2, gs2, bes2, ms2, vs2,
           Wo, bo, Wb, bb):
    f32 = _f32
    i32 = jnp.int32
    pad = _EP - _E
    r2 = jnp.concatenate([edge_index[0], jnp.zeros((pad,), i32)]
                         ).reshape(_EP_ROWS, 128)
    c2 = jnp.concatenate([edge_index[1], jnp.zeros((pad,), i32)]
                         ).reshape(_EP_ROWS, 128)
    w2 = jnp.concatenate([edge_weight, jnp.zeros((pad,), f32)]
                         ).reshape(_EP_ROWS, 128)
    degp = _deg_call(c2, w2).reshape(_NC, _N2)[:, :_N].T

    dis2d, hp1 = _tcb_call(degp, x, W1)
    hp1s = jnp.stack([hp1[:, :16], hp1[:, 16:]])
    agg1 = _conv_call(r2, c2, w2, hp1s)[:, :_N]

    hp2 = _tcmid_call(agg1, hp1, dis2d,
                      b1.reshape(1, _H), g1.reshape(1, _H), be1.reshape(1, _H),
                      m1.reshape(1, _H), v1.reshape(1, _H), W2,
                      matmul=True, name="tc_mid1")
    hp2s = jnp.stack([hp2[:, :16], hp2[:, 16:]])
    agg2 = _conv_call(r2, c2, w2, hp2s)[:, :_N]

    z2 = _tcmid_call(agg2, hp2, dis2d,
                     b2.reshape(1, _H), g2.reshape(1, _H), be2.reshape(1, _H),
                     m2.reshape(1, _H), v2.reshape(1, _H), W2,
                     matmul=False, name="tc_mid2")

    z2p = jnp.concatenate([z2, jnp.zeros((_NP - _N, _H), f32)])
    bp = jnp.concatenate([batch, jnp.zeros((_NP - _N,), i32)]
                         ).reshape(_NPG, 128)
    vp = jnp.concatenate([jnp.ones((_N,), f32), jnp.zeros((_NP - _N,), f32)]
                         ).reshape(_NPG, 128)
    sums_o, cnts_o = _pool_call(z2p, bp, vp)
    sums = sums_o.reshape(_NC, _B2S, _H)[:, :_B]
    cnts3 = cnts_o.reshape(_NC, _B2)[:, :_B].reshape(_NC, _B, 1)

    orange, blue = _tch_call(
        sums, cnts3, global_features,
        Ws1, bs1.reshape(1, 128), gs1.reshape(1, 128), bes1.reshape(1, 128),
        ms1.reshape(1, 128), vs1.reshape(1, 128),
        Ws2, bs2.reshape(1, 64), gs2.reshape(1, 64), bes2.reshape(1, 64),
        ms2.reshape(1, 64), vs2.reshape(1, 64),
        Wo, bo.reshape(1, 1), Wb, bb.reshape(1, 1))
    return (orange, blue)


# speedup vs baseline: 34.1896x; 2.1151x over previous
import jax
import jax.numpy as jnp
from jax import lax
from jax.experimental import pallas as pl
from jax.experimental.pallas import tpu as pltpu
from jax.experimental.pallas import tpu_sc as plsc

_N = 100000
_E = 3200000
_B = 10000
_H = 32
_NC = 2
_NS = 16
_EP_ROWS = 25088
_EP = _EP_ROWS * 128
_N2 = 100096
_B2 = 10112
_NP = 102400
_NPG = _NP // 128

_mesh = plsc.VectorSubcoreMesh(core_axis_name="c", subcore_axis_name="s")
_f32 = jnp.float32
_SC_PARAMS = pltpu.CompilerParams(use_tc_tiling_on_sc=False)


def _deg_body(c_hbm, w_hbm, out_hbm, acc, zbuf, cstage, wstage):
    core = lax.axis_index("c")
    sub = lax.axis_index("s")
    wid = core * _NS + sub
    n_sl = _N2 // _NS
    zv = jnp.zeros((16,), _f32)

    @pl.loop(0, 64)
    def _(i):
        zbuf[pl.ds(i * 16, 16)] = zv

    @pl.loop(0, 6)
    def _(k):
        pltpu.sync_copy(zbuf, acc.at[pl.ds(sub * n_sl + k * 1024, 1024)])

    pltpu.sync_copy(zbuf.at[pl.ds(0, 112)],
                    acc.at[pl.ds(sub * n_sl + 6144, 112)])
    plsc.subcore_barrier()
    rows_per_tile = _EP_ROWS // (_NC * _NS)

    @pl.loop(0, rows_per_tile // 8)
    def _(g):
        row0 = wid * rows_per_tile + g * 8
        pltpu.sync_copy(c_hbm.at[pl.ds(row0, 8)], cstage)
        pltpu.sync_copy(w_hbm.at[pl.ds(row0, 8)], wstage)
        for j in range(8):
            pltpu.sync_copy(wstage.at[j], acc.at[cstage.at[j]], add=True)

    plsc.subcore_barrier()
    base = core * _N2 + sub * n_sl

    @pl.loop(0, 6)
    def _(k):
        pltpu.sync_copy(acc.at[pl.ds(sub * n_sl + k * 1024, 1024)], zbuf)
        pltpu.sync_copy(zbuf, out_hbm.at[pl.ds(base + k * 1024, 1024)])

    pltpu.sync_copy(acc.at[pl.ds(sub * n_sl + 6144, 112)],
                    zbuf.at[pl.ds(0, 112)])
    pltpu.sync_copy(zbuf.at[pl.ds(0, 112)],
                    out_hbm.at[pl.ds(base + 6144, 112)])


def _deg_call(c2, w2):
    return pl.kernel(
        _deg_body,
        out_type=jax.ShapeDtypeStruct((_NC * _N2,), _f32),
        mesh=_mesh,
        scratch_types=[
            pltpu.VMEM_SHARED((_N2,), _f32),
            pltpu.VMEM((1024,), _f32),
            pltpu.VMEM((8, 128), jnp.int32),
            pltpu.VMEM((8, 128), _f32),
        ],
        name="sc_deg",
        compiler_params=_SC_PARAMS,
    )(c2, w2)


_SROWS = 4
_SEDGES = _SROWS * 128
_NSTAGES = (_EP_ROWS // _NS) // _SROWS


def _conv_body(r_hbm, c_hbm, w_hbm, hp_hbm, out_hbm,
               acc,
               rst0, wst0, rows0, rst1, wst1, rows1,
               cstA, cstB, cstC, cstD,
               sem_i0, sem_g0, sem_s0, sem_i1, sem_g1, sem_s1):
    core = lax.axis_index("c")
    sub = lax.axis_index("s")
    n_sl = _N2 // _NS
    zv = jnp.zeros((1, 16), _f32)

    @pl.loop(0, _SEDGES)
    def _(i):
        rows0[pl.ds(i, 1), :] = zv

    @pl.loop(0, 12)
    def _(k):
        pltpu.sync_copy(rows0, acc.at[pl.ds(sub * n_sl + k * 512, 512)])

    pltpu.sync_copy(rows0.at[pl.ds(0, 112)],
                    acc.at[pl.ds(sub * n_sl + 6144, 112)])
    plsc.subcore_barrier()
    row_base = sub * (_EP_ROWS // _NS)

    def fire_idx(s, rst, cst, wst, sem):
        row0 = row_base + s * _SROWS
        pltpu.async_copy(r_hbm.at[pl.ds(row0, _SROWS)], rst, sem)
        pltpu.async_copy(c_hbm.at[pl.ds(row0, _SROWS)], cst, sem)
        pltpu.async_copy(w_hbm.at[pl.ds(row0, _SROWS)], wst, sem)

    def wait_idx(rst, cst, wst, sem):
        pltpu.make_async_copy(r_hbm.at[pl.ds(0, _SROWS)], rst, sem).wait()
        pltpu.make_async_copy(c_hbm.at[pl.ds(0, _SROWS)], cst, sem).wait()
        pltpu.make_async_copy(w_hbm.at[pl.ds(0, _SROWS)], wst, sem).wait()

    def fire_gathers(rst, rows, sem):
        for j in range(_SROWS):
            pltpu.async_copy(hp_hbm.at[core].at[rst.at[j]],
                             rows.at[pl.ds(j * 128, 128), :], sem)

    def wait_gathers(rows, sem):
        pltpu.make_async_copy(hp_hbm.at[core, pl.ds(0, _SEDGES), :], rows,
                              sem).wait()

    def wait_scatters(rows, sem):
        pltpu.make_async_copy(rows, acc.at[pl.ds(0, _SEDGES)], sem).wait()

    def scale_and_scatter(cst, wst, rows, sem):
        for j in range(_SROWS):
            @pl.loop(0, 8)
            def _(q):
                w16 = wst.at[j][pl.ds(q * 16, 16)]
                for e in range(16):
                    idx = j * 128 + q * 16 + e
                    rows[pl.ds(idx, 1), :] = rows[pl.ds(idx, 1), :] * w16[e]

            pltpu.async_copy(rows.at[pl.ds(j * 128, 128), :],
                             acc.at[cst.at[j]], sem, add=True)

    fire_idx(0, rst0, cstA, wst0, sem_i0)
    wait_idx(rst0, cstA, wst0, sem_i0)
    fire_gathers(rst0, rows0, sem_g0)
    fire_idx(1, rst1, cstB, wst1, sem_i1)

    def process(s, rstP, wstP, cstS, rowsP, sem_iP, sem_gP, sem_sP,
                rstQ, wstQ, cstQn, rowsQ, sem_iQ, sem_gQ, sem_sQ, cstNext):
        wait_gathers(rowsP, sem_gP)

        @pl.when(s > 0)
        def _():
            wait_scatters(rowsQ, sem_sQ)

        @pl.when(s + 1 < _NSTAGES)
        def _():
            wait_idx(rstQ, cstQn, wstQ, sem_iQ)
            fire_gathers(rstQ, rowsQ, sem_gQ)

        scale_and_scatter(cstS, wstP, rowsP, sem_sP)

        @pl.when(s + 2 < _NSTAGES)
        def _():
            fire_idx(s + 2, rstP, cstNext, wstP, sem_iP)

    @pl.loop(0, _NSTAGES // 4)
    def _(m):
        s = m * 4
        process(s, rst0, wst0, cstA, rows0, sem_i0, sem_g0, sem_s0,
                rst1, wst1, cstB, rows1, sem_i1, sem_g1, sem_s1, cstC)
        process(s + 1, rst1, wst1, cstB, rows1, sem_i1, sem_g1, sem_s1,
                rst0, wst0, cstC, rows0, sem_i0, sem_g0, sem_s0, cstD)
        process(s + 2, rst0, wst0, cstC, rows0, sem_i0, sem_g0, sem_s0,
                rst1, wst1, cstD, rows1, sem_i1, sem_g1, sem_s1, cstA)
        process(s + 3, rst1, wst1, cstD, rows1, sem_i1, sem_g1, sem_s1,
                rst0, wst0, cstA, rows0, sem_i0, sem_g0, sem_s0, cstB)

    wait_scatters(rows1, sem_s1)
    plsc.subcore_barrier()

    @pl.loop(0, 12)
    def _(k):
        pltpu.sync_copy(acc.at[pl.ds(sub * n_sl + k * 512, 512)], rows0)
        pltpu.sync_copy(rows0, out_hbm.at[core, pl.ds(sub * n_sl + k * 512,
                                                      512)])

    pltpu.sync_copy(acc.at[pl.ds(sub * n_sl + 6144, 112)],
                    rows0.at[pl.ds(0, 112)])
    pltpu.sync_copy(rows0.at[pl.ds(0, 112)],
                    out_hbm.at[core, pl.ds(sub * n_sl + 6144, 112)])


def _conv_call(r2, c2, w2, hps):
    return pl.kernel(
        _conv_body,
        out_type=jax.ShapeDtypeStruct((_NC, _N2, 16), _f32),
        mesh=_mesh,
        scratch_types=[
            pltpu.VMEM_SHARED((_N2, 16), _f32),
            pltpu.VMEM((_SROWS, 128), jnp.int32),
            pltpu.VMEM((_SROWS, 128), _f32),
            pltpu.VMEM((_SEDGES, 16), _f32),
            pltpu.VMEM((_SROWS, 128), jnp.int32),
            pltpu.VMEM((_SROWS, 128), _f32),
            pltpu.VMEM((_SEDGES, 16), _f32),
            pltpu.VMEM((_SROWS, 128), jnp.int32),
            pltpu.VMEM((_SROWS, 128), jnp.int32),
            pltpu.VMEM((_SROWS, 128), jnp.int32),
            pltpu.VMEM((_SROWS, 128), jnp.int32),
            pltpu.SemaphoreType.DMA,
            pltpu.SemaphoreType.DMA,
            pltpu.SemaphoreType.DMA,
            pltpu.SemaphoreType.DMA,
            pltpu.SemaphoreType.DMA,
            pltpu.SemaphoreType.DMA,
        ],
        name="sc_conv",
        compiler_params=_SC_PARAMS,
    )(r2, c2, w2, hps)


_B2S = 10240


def _pool_body(z_hbm, b_hbm, v_hbm, sums_hbm, cnts_hbm,
               accS, accC, zbuf, zbufc, zstage, bstage, vstage):
    core = lax.axis_index("c")
    sub = lax.axis_index("s")
    wid = core * _NS + sub
    b_sl = _B2S // _NS
    b_sl2 = _B2 // _NS
    zv = jnp.zeros((16,), _f32)

    @pl.loop(0, 640)
    def _(i):
        zbuf[pl.ds(i, 1), :] = jnp.zeros((1, _H), _f32)

    @pl.loop(0, 40)
    def _(i):
        zbufc[pl.ds(i * 16, 16)] = zv

    pltpu.sync_copy(zbuf, accS.at[pl.ds(sub * b_sl, b_sl)])
    pltpu.sync_copy(zbufc.at[pl.ds(0, b_sl2)],
                    accC.at[pl.ds(sub * b_sl2, b_sl2)])
    plsc.subcore_barrier()
    groups_per_tile = _NPG // (_NC * _NS)

    @pl.loop(0, groups_per_tile)
    def _(g):
        grp = wid * groups_per_tile + g
        pltpu.sync_copy(z_hbm.at[pl.ds(grp * 128, 128)], zstage)
        pltpu.sync_copy(b_hbm.at[pl.ds(grp, 1)], bstage)
        pltpu.sync_copy(v_hbm.at[pl.ds(grp, 1)], vstage)
        pltpu.sync_copy(zstage, accS.at[bstage.at[0]], add=True)
        pltpu.sync_copy(vstage.at[0], accC.at[bstage.at[0]], add=True)

    plsc.subcore_barrier()
    pltpu.sync_copy(accS.at[pl.ds(sub * b_sl, b_sl)], zbuf)
    pltpu.sync_copy(zbuf, sums_hbm.at[pl.ds(core * _B2S + sub * b_sl, b_sl)])
    pltpu.sync_copy(accC.at[pl.ds(sub * b_sl2, b_sl2)],
                    zbufc.at[pl.ds(0, b_sl2)])
    pltpu.sync_copy(zbufc.at[pl.ds(0, b_sl2)],
                    cnts_hbm.at[pl.ds(core * _B2 + sub * b_sl2, b_sl2)])


def _pool_call(z2p, bp, vp):
    return pl.kernel(
        _pool_body,
        out_type=(jax.ShapeDtypeStruct((_NC * _B2S, _H), _f32),
                  jax.ShapeDtypeStruct((_NC * _B2,), _f32)),
        mesh=_mesh,
        scratch_types=[
            pltpu.VMEM_SHARED((_B2S, _H), _f32),
            pltpu.VMEM_SHARED((_B2,), _f32),
            pltpu.VMEM((640, _H), _f32),
            pltpu.VMEM((640,), _f32),
            pltpu.VMEM((128, _H), _f32),
            pltpu.VMEM((1, 128), jnp.int32),
            pltpu.VMEM((1, 128), _f32),
        ],
        name="sc_pool",
        compiler_params=_SC_PARAMS,
    )(z2p, bp, vp)


_BLK = 10000


def _tcb_body(dp_ref, x_ref, w1_ref, dis_ref, hp_ref):
    deg = 1.0 + dp_ref[:, 0] + dp_ref[:, 1]
    dis = lax.rsqrt(deg)
    h = jnp.dot(x_ref[...], w1_ref[...], preferred_element_type=_f32)
    hp_ref[...] = h * dis[:, None]
    dis_ref[...] = dis[:, None]


def _tcb_call(degp, x, W1):
    grid = (_N // _BLK,)
    return pl.pallas_call(
        _tcb_body,
        grid=grid,
        in_specs=[
            pl.BlockSpec((_BLK, _NC), lambda i: (i, 0)),
            pl.BlockSpec((_BLK, 13), lambda i: (i, 0)),
            pl.BlockSpec((13, _H), lambda i: (0, 0)),
        ],
        out_specs=[
            pl.BlockSpec((_BLK, 1), lambda i: (i, 0)),
            pl.BlockSpec((_BLK, _H), lambda i: (i, 0)),
        ],
        out_shape=[
            jax.ShapeDtypeStruct((_N, 1), _f32),
            jax.ShapeDtypeStruct((_N, _H), _f32),
        ],
        name="tc_prep",
    )(degp, x, W1)


def _tcmid_body(a_ref, hp_ref, dis_ref, b_ref, g_ref, be_ref, m_ref, v_ref,
                w2_ref, out_ref, *, matmul):
    agg = jnp.concatenate([a_ref[0], a_ref[1]], axis=-1)
    dis = dis_ref[...]
    conv = dis * (agg + hp_ref[...]) + b_ref[...]
    z = jnp.maximum(
        (conv - m_ref[...]) * lax.rsqrt(v_ref[...] + 1e-5) * g_ref[...]
        + be_ref[...], 0.0)
    if matmul:
        h2 = jnp.dot(z, w2_ref[...], preferred_element_type=_f32)
        out_ref[...] = h2 * dis
    else:
        out_ref[...] = z


def _tcmid_call(agg, hp, dis2d, b, g, be, m, v, W2, matmul, name):
    import functools
    grid = (_N // _BLK,)
    body = functools.partial(_tcmid_body, matmul=matmul)
    return pl.pallas_call(
        body,
        grid=grid,
        in_specs=[
            pl.BlockSpec((_NC, _BLK, 16), lambda i: (0, i, 0)),
            pl.BlockSpec((_BLK, _H), lambda i: (i, 0)),
            pl.BlockSpec((_BLK, 1), lambda i: (i, 0)),
            pl.BlockSpec((1, _H), lambda i: (0, 0)),
            pl.BlockSpec((1, _H), lambda i: (0, 0)),
            pl.BlockSpec((1, _H), lambda i: (0, 0)),
            pl.BlockSpec((1, _H), lambda i: (0, 0)),
            pl.BlockSpec((1, _H), lambda i: (0, 0)),
            pl.BlockSpec((_H, _H), lambda i: (0, 0)),
        ],
        out_specs=pl.BlockSpec((_BLK, _H), lambda i: (i, 0)),
        out_shape=jax.ShapeDtypeStruct((_N, _H), _f32),
        name=name,
    )(agg, hp, dis2d, b, g, be, m, v, W2)


def _tch_body(sums_ref, cnts_ref, gf_ref,
              ws1_ref, bs1_ref, gs1_ref, bes1_ref, ms1_ref, vs1_ref,
              ws2_ref, bs2_ref, gs2_ref, bes2_ref, ms2_ref, vs2_ref,
              wo_ref, bo_ref, wb_ref, bb_ref, or_ref, bl_ref):
    sums = sums_ref[0] + sums_ref[1]
    cnts = cnts_ref[0] + cnts_ref[1]
    ge = sums / jnp.clip(cnts, 1.0)
    comb = jnp.concatenate([ge, gf_ref[...]], axis=1)
    s = jnp.dot(comb, ws1_ref[...], preferred_element_type=_f32) + bs1_ref[...]
    s = jnp.maximum(
        (s - ms1_ref[...]) * lax.rsqrt(vs1_ref[...] + 1e-5) * gs1_ref[...]
        + bes1_ref[...], 0.0)
    s = jnp.dot(s, ws2_ref[...], preferred_element_type=_f32) + bs2_ref[...]
    s = jnp.maximum(
        (s - ms2_ref[...]) * lax.rsqrt(vs2_ref[...] + 1e-5) * gs2_ref[...]
        + bes2_ref[...], 0.0)
    or_ref[...] = jax.nn.sigmoid(
        jnp.dot(s, wo_ref[...], preferred_element_type=_f32) + bo_ref[...])
    bl_ref[...] = jax.nn.sigmoid(
        jnp.dot(s, wb_ref[...], preferred_element_type=_f32) + bb_ref[...])


def _tch_call(sums, cnts3, gf, Ws1, bs1, gs1, bes1, ms1, vs1,
              Ws2, bs2, gs2, bes2, ms2, vs2, Wo, bo, Wb, bb):
    return pl.pallas_call(
        _tch_body,
        out_shape=[
            jax.ShapeDtypeStruct((_B, 1), _f32),
            jax.ShapeDtypeStruct((_B, 1), _f32),
        ],
        name="tc_head",
    )(sums, cnts3, gf, Ws1, bs1, gs1, bes1, ms1, vs1,
      Ws2, bs2, gs2, bes2, ms2, vs2, Wo, bo, Wb, bb)


def kernel(x, edge_index, edge_weight, batch, global_features,
           W1, b1, g1, be1, m1, v1,
           W2, b2, g2, be2, m2, v2,
           Ws1, bs1, gs1, bes1, ms1, vs1,
           Ws2, bs2, gs2, bes2, ms2, vs2,
           Wo, bo, Wb, bb):
    f32 = _f32
    i32 = jnp.int32
    pad = _EP - _E
    r2 = jnp.concatenate([edge_index[0], jnp.zeros((pad,), i32)]
                         ).reshape(_EP_ROWS, 128)
    c2 = jnp.concatenate([edge_index[1], jnp.zeros((pad,), i32)]
                         ).reshape(_EP_ROWS, 128)
    w2 = jnp.concatenate([edge_weight, jnp.zeros((pad,), f32)]
                         ).reshape(_EP_ROWS, 128)
    degp = _deg_call(c2, w2).reshape(_NC, _N2)[:, :_N].T

    dis2d, hp1 = _tcb_call(degp, x, W1)
    hp1s = jnp.stack([hp1[:, :16], hp1[:, 16:]])
    agg1 = _conv_call(r2, c2, w2, hp1s)[:, :_N]

    hp2 = _tcmid_call(agg1, hp1, dis2d,
                      b1.reshape(1, _H), g1.reshape(1, _H), be1.reshape(1, _H),
                      m1.reshape(1, _H), v1.reshape(1, _H), W2,
                      matmul=True, name="tc_mid1")
    hp2s = jnp.stack([hp2[:, :16], hp2[:, 16:]])
    agg2 = _conv_call(r2, c2, w2, hp2s)[:, :_N]

    z2 = _tcmid_call(agg2, hp2, dis2d,
                     b2.reshape(1, _H), g2.reshape(1, _H), be2.reshape(1, _H),
                     m2.reshape(1, _H), v2.reshape(1, _H), W2,
                     matmul=False, name="tc_mid2")

    z2p = jnp.concatenate([z2, jnp.zeros((_NP - _N, _H), f32)])
    bp = jnp.concatenate([batch, jnp.zeros((_NP - _N,), i32)]
                         ).reshape(_NPG, 128)
    vp = jnp.concatenate([jnp.ones((_N,), f32), jnp.zeros((_NP - _N,), f32)]
                         ).reshape(_NPG, 128)
    sums_o, cnts_o = _pool_call(z2p, bp, vp)
    sums = sums_o.reshape(_NC, _B2S, _H)[:, :_B]
    cnts3 = cnts_o.reshape(_NC, _B2)[:, :_B].reshape(_NC, _B, 1)

    orange, blue = _tch_call(
        sums, cnts3, global_features,
        Ws1, bs1.reshape(1, 128), gs1.reshape(1, 128), bes1.reshape(1, 128),
        ms1.reshape(1, 128), vs1.reshape(1, 128),
        Ws2, bs2.reshape(1, 64), gs2.reshape(1, 64), bes2.reshape(1, 64),
        ms2.reshape(1, 64), vs2.reshape(1, 64),
        Wo, bo.reshape(1, 1), Wb, bb.reshape(1, 1))
    return (orange, blue)


# speedup vs baseline: 38.8573x; 1.1365x over previous
import jax
import jax.numpy as jnp
from jax import lax
from jax.experimental import pallas as pl
from jax.experimental.pallas import tpu as pltpu
from jax.experimental.pallas import tpu_sc as plsc

_N = 100000
_E = 3200000
_B = 10000
_H = 32
_NC = 2
_NS = 16
_EP_ROWS = 25088
_EP = _EP_ROWS * 128
_N2 = 100096
_B2 = 10112
_NP = 102400
_NPG = _NP // 128

_mesh = plsc.VectorSubcoreMesh(core_axis_name="c", subcore_axis_name="s")
_f32 = jnp.float32
_SC_PARAMS = pltpu.CompilerParams(use_tc_tiling_on_sc=False)


def _deg_body(c_hbm, w_hbm, out_hbm, acc, zbuf, cstage, wstage):
    core = lax.axis_index("c")
    sub = lax.axis_index("s")
    wid = core * _NS + sub
    n_sl = _N2 // _NS
    zv = jnp.zeros((16,), _f32)

    @pl.loop(0, 64)
    def _(i):
        zbuf[pl.ds(i * 16, 16)] = zv

    @pl.loop(0, 6)
    def _(k):
        pltpu.sync_copy(zbuf, acc.at[pl.ds(sub * n_sl + k * 1024, 1024)])

    pltpu.sync_copy(zbuf.at[pl.ds(0, 112)],
                    acc.at[pl.ds(sub * n_sl + 6144, 112)])
    plsc.subcore_barrier()
    rows_per_tile = _EP_ROWS // (_NC * _NS)

    @pl.loop(0, rows_per_tile // 8)
    def _(g):
        row0 = wid * rows_per_tile + g * 8
        pltpu.sync_copy(c_hbm.at[pl.ds(row0, 8)], cstage)
        pltpu.sync_copy(w_hbm.at[pl.ds(row0, 8)], wstage)
        for j in range(8):
            pltpu.sync_copy(wstage.at[j], acc.at[cstage.at[j]], add=True)

    plsc.subcore_barrier()
    base = core * _N2 + sub * n_sl

    @pl.loop(0, 6)
    def _(k):
        pltpu.sync_copy(acc.at[pl.ds(sub * n_sl + k * 1024, 1024)], zbuf)
        pltpu.sync_copy(zbuf, out_hbm.at[pl.ds(base + k * 1024, 1024)])

    pltpu.sync_copy(acc.at[pl.ds(sub * n_sl + 6144, 112)],
                    zbuf.at[pl.ds(0, 112)])
    pltpu.sync_copy(zbuf.at[pl.ds(0, 112)],
                    out_hbm.at[pl.ds(base + 6144, 112)])


def _deg_call(c2, w2):
    return pl.kernel(
        _deg_body,
        out_type=jax.ShapeDtypeStruct((_NC * _N2,), _f32),
        mesh=_mesh,
        scratch_types=[
            pltpu.VMEM_SHARED((_N2,), _f32),
            pltpu.VMEM((1024,), _f32),
            pltpu.VMEM((8, 128), jnp.int32),
            pltpu.VMEM((8, 128), _f32),
        ],
        name="sc_deg",
        compiler_params=_SC_PARAMS,
    )(c2, w2)


_SROWS = 4
_SEDGES = _SROWS * 128
_NSTAGES = (_EP_ROWS // _NS) // _SROWS


def _conv_body(r_hbm, c_hbm, w_hbm, hp_hbm, out_hbm,
               acc,
               rst0, wst0, rows0, rst1, wst1, rows1,
               cstA, cstB, cstC, cstD,
               sem_i0, sem_g0, sem_s0, sem_i1, sem_g1, sem_s1):
    core = lax.axis_index("c")
    sub = lax.axis_index("s")
    n_sl = _N2 // _NS
    zv = jnp.zeros((1, 16), _f32)

    @pl.loop(0, _SEDGES)
    def _(i):
        rows0[pl.ds(i, 1), :] = zv

    @pl.loop(0, 12)
    def _(k):
        pltpu.sync_copy(rows0, acc.at[pl.ds(sub * n_sl + k * 512, 512)])

    pltpu.sync_copy(rows0.at[pl.ds(0, 112)],
                    acc.at[pl.ds(sub * n_sl + 6144, 112)])
    plsc.subcore_barrier()
    row_base = sub * (_EP_ROWS // _NS)

    def fire_idx(s, rst, cst, wst, sem):
        row0 = row_base + s * _SROWS
        pltpu.async_copy(r_hbm.at[pl.ds(row0, _SROWS)], rst, sem)
        pltpu.async_copy(c_hbm.at[pl.ds(row0, _SROWS)], cst, sem)
        pltpu.async_copy(w_hbm.at[pl.ds(row0, _SROWS)], wst, sem)

    def wait_idx(rst, cst, wst, sem):
        pltpu.make_async_copy(r_hbm.at[pl.ds(0, _SROWS)], rst, sem).wait()
        pltpu.make_async_copy(c_hbm.at[pl.ds(0, _SROWS)], cst, sem).wait()
        pltpu.make_async_copy(w_hbm.at[pl.ds(0, _SROWS)], wst, sem).wait()

    def fire_gathers(rst, rows, sem):
        for j in range(_SROWS):
            pltpu.async_copy(hp_hbm.at[core].at[rst.at[j]],
                             rows.at[pl.ds(j * 128, 128), :], sem)

    def wait_gathers(rows, sem):
        pltpu.make_async_copy(hp_hbm.at[core, pl.ds(0, _SEDGES), :], rows,
                              sem).wait()

    def wait_scatters(rows, sem):
        pltpu.make_async_copy(rows, acc.at[pl.ds(0, _SEDGES)], sem).wait()

    def scale_and_scatter(cst, wst, rows, sem):
        for j in range(_SROWS):
            @pl.loop(0, 8)
            def _(q):
                w16 = wst.at[j][pl.ds(q * 16, 16)]
                for e in range(16):
                    idx = j * 128 + q * 16 + e
                    rows[pl.ds(idx, 1), :] = rows[pl.ds(idx, 1), :] * w16[e]

            pltpu.async_copy(rows.at[pl.ds(j * 128, 128), :],
                             acc.at[cst.at[j]], sem, add=True)

    fire_idx(0, rst0, cstA, wst0, sem_i0)
    wait_idx(rst0, cstA, wst0, sem_i0)
    fire_gathers(rst0, rows0, sem_g0)
    fire_idx(1, rst1, cstB, wst1, sem_i1)

    def process(s, rstP, wstP, cstS, rowsP, sem_iP, sem_gP, sem_sP,
                rstQ, wstQ, cstQn, rowsQ, sem_iQ, sem_gQ, sem_sQ, cstNext):
        wait_gathers(rowsP, sem_gP)

        @pl.when(s > 0)
        def _():
            wait_scatters(rowsQ, sem_sQ)

        @pl.when(s + 1 < _NSTAGES)
        def _():
            wait_idx(rstQ, cstQn, wstQ, sem_iQ)
            fire_gathers(rstQ, rowsQ, sem_gQ)

        scale_and_scatter(cstS, wstP, rowsP, sem_sP)

        @pl.when(s + 2 < _NSTAGES)
        def _():
            fire_idx(s + 2, rstP, cstNext, wstP, sem_iP)

    @pl.loop(0, _NSTAGES // 4)
    def _(m):
        s = m * 4
        process(s, rst0, wst0, cstA, rows0, sem_i0, sem_g0, sem_s0,
                rst1, wst1, cstB, rows1, sem_i1, sem_g1, sem_s1, cstC)
        process(s + 1, rst1, wst1, cstB, rows1, sem_i1, sem_g1, sem_s1,
                rst0, wst0, cstC, rows0, sem_i0, sem_g0, sem_s0, cstD)
        process(s + 2, rst0, wst0, cstC, rows0, sem_i0, sem_g0, sem_s0,
                rst1, wst1, cstD, rows1, sem_i1, sem_g1, sem_s1, cstA)
        process(s + 3, rst1, wst1, cstD, rows1, sem_i1, sem_g1, sem_s1,
                rst0, wst0, cstA, rows0, sem_i0, sem_g0, sem_s0, cstB)

    wait_scatters(rows1, sem_s1)
    plsc.subcore_barrier()

    @pl.loop(0, 12)
    def _(k):
        pltpu.sync_copy(acc.at[pl.ds(sub * n_sl + k * 512, 512)], rows0)
        pltpu.sync_copy(rows0, out_hbm.at[core, pl.ds(sub * n_sl + k * 512,
                                                      512)])

    pltpu.sync_copy(acc.at[pl.ds(sub * n_sl + 6144, 112)],
                    rows0.at[pl.ds(0, 112)])
    pltpu.sync_copy(rows0.at[pl.ds(0, 112)],
                    out_hbm.at[core, pl.ds(sub * n_sl + 6144, 112)])


def _conv_call(r2, c2, w2, hps):
    return pl.kernel(
        _conv_body,
        out_type=jax.ShapeDtypeStruct((_NC, _N2, 16), _f32),
        mesh=_mesh,
        scratch_types=[
            pltpu.VMEM_SHARED((_N2, 16), _f32),
            pltpu.VMEM((_SROWS, 128), jnp.int32),
            pltpu.VMEM((_SROWS, 128), _f32),
            pltpu.VMEM((_SEDGES, 16), _f32),
            pltpu.VMEM((_SROWS, 128), jnp.int32),
            pltpu.VMEM((_SROWS, 128), _f32),
            pltpu.VMEM((_SEDGES, 16), _f32),
            pltpu.VMEM((_SROWS, 128), jnp.int32),
            pltpu.VMEM((_SROWS, 128), jnp.int32),
            pltpu.VMEM((_SROWS, 128), jnp.int32),
            pltpu.VMEM((_SROWS, 128), jnp.int32),
            pltpu.SemaphoreType.DMA,
            pltpu.SemaphoreType.DMA,
            pltpu.SemaphoreType.DMA,
            pltpu.SemaphoreType.DMA,
            pltpu.SemaphoreType.DMA,
            pltpu.SemaphoreType.DMA,
        ],
        name="sc_conv",
        compiler_params=_SC_PARAMS,
    )(r2, c2, w2, hps)


_B2S = 10240


def _pool_body(z_hbm, b_hbm, sums_hbm, cnts_hbm,
               accS, accC, zbuf, zbufc, ones, zstage, bstage):
    core = lax.axis_index("c")
    sub = lax.axis_index("s")
    wid = core * _NS + sub
    b_sl = _B2S // _NS
    zv = jnp.zeros((16,), _f32)
    ov = jnp.ones((1, 16), _f32)

    @pl.loop(0, 640)
    def _(i):
        zbuf[pl.ds(i, 1), :] = jnp.zeros((1, _H), _f32)

    @pl.loop(0, 40)
    def _(i):
        zbufc[pl.ds(i * 16, 16)] = zv

    for k in range(8):
        ones[pl.ds(0, 1), pl.ds(k * 16, 16)] = ov

    pltpu.sync_copy(zbuf, accS.at[pl.ds(sub * b_sl, b_sl)])
    pltpu.sync_copy(zbufc, accC.at[pl.ds(sub * b_sl, b_sl)])
    plsc.subcore_barrier()
    groups_per_tile = _NPG // (_NC * _NS)

    @pl.loop(0, groups_per_tile)
    def _(g):
        grp = wid * groups_per_tile + g
        pltpu.sync_copy(z_hbm.at[pl.ds(grp * 128, 128)], zstage)
        pltpu.sync_copy(b_hbm.at[pl.ds(grp, 1)], bstage)
        pltpu.sync_copy(zstage, accS.at[bstage.at[0]], add=True)
        pltpu.sync_copy(ones.at[0], accC.at[bstage.at[0]], add=True)

    plsc.subcore_barrier()
    pltpu.sync_copy(accS.at[pl.ds(sub * b_sl, b_sl)], zbuf)
    pltpu.sync_copy(zbuf, sums_hbm.at[pl.ds(core * _B2S + sub * b_sl, b_sl)])
    pltpu.sync_copy(accC.at[pl.ds(sub * b_sl, b_sl)], zbufc)
    pltpu.sync_copy(zbufc,
                    cnts_hbm.at[pl.ds(core * _B2S + sub * b_sl, b_sl)])


def _pool_call(z2p, bp):
    return pl.kernel(
        _pool_body,
        out_type=(jax.ShapeDtypeStruct((_NC * _B2S, _H), _f32),
                  jax.ShapeDtypeStruct((_NC * _B2S,), _f32)),
        mesh=_mesh,
        scratch_types=[
            pltpu.VMEM_SHARED((_B2S, _H), _f32),
            pltpu.VMEM_SHARED((_B2S,), _f32),
            pltpu.VMEM((640, _H), _f32),
            pltpu.VMEM((640,), _f32),
            pltpu.VMEM((1, 128), _f32),
            pltpu.VMEM((128, _H), _f32),
            pltpu.VMEM((1, 128), jnp.int32),
        ],
        name="sc_pool",
        compiler_params=_SC_PARAMS,
    )(z2p, bp)


_BLK = 5000


_ER = _E // 128
_PBLK = 1792


def _pad_body(ei_ref, w_ref, r_ref, c_ref, w2_ref):
    i = pl.program_id(0)
    rid = lax.broadcasted_iota(jnp.int32, (_PBLK, 128), 0) + i * _PBLK
    m = rid < _ER
    r_ref[...] = jnp.where(m, ei_ref[0], 0)
    c_ref[...] = jnp.where(m, ei_ref[1], 0)
    w2_ref[...] = jnp.where(m, w_ref[...], 0.0)


def _pad_call(ei3, w3):
    return pl.pallas_call(
        _pad_body,
        grid=(_EP_ROWS // _PBLK,),
        in_specs=[
            pl.BlockSpec((2, _PBLK, 128), lambda i: (0, i, 0)),
            pl.BlockSpec((_PBLK, 128), lambda i: (i, 0)),
        ],
        out_specs=[
            pl.BlockSpec((_PBLK, 128), lambda i: (i, 0)),
            pl.BlockSpec((_PBLK, 128), lambda i: (i, 0)),
            pl.BlockSpec((_PBLK, 128), lambda i: (i, 0)),
        ],
        out_shape=[
            jax.ShapeDtypeStruct((_EP_ROWS, 128), jnp.int32),
            jax.ShapeDtypeStruct((_EP_ROWS, 128), jnp.int32),
            jax.ShapeDtypeStruct((_EP_ROWS, 128), _f32),
        ],
        name="tc_pad",
    )(ei3, w3)


def _tcb_body(dpa_ref, dpb_ref, x_ref, w1_ref, dis_ref, hp_ref):
    deg = 1.0 + dpa_ref[:, 0] + dpb_ref[:, 0]
    dis = lax.rsqrt(deg)
    h = jnp.dot(x_ref[...], w1_ref[...], preferred_element_type=_f32)
    hp = h * dis[:, None]
    hp_ref[0] = hp[:, :16]
    hp_ref[1] = hp[:, 16:]
    dis_ref[...] = dis[:, None]


def _tcb_call(dpa, dpb, x, W1):
    grid = (_N // _BLK,)
    return pl.pallas_call(
        _tcb_body,
        grid=grid,
        in_specs=[
            pl.BlockSpec((_BLK, 1), lambda i: (i, 0)),
            pl.BlockSpec((_BLK, 1), lambda i: (i, 0)),
            pl.BlockSpec((_BLK, 13), lambda i: (i, 0)),
            pl.BlockSpec((13, _H), lambda i: (0, 0)),
        ],
        out_specs=[
            pl.BlockSpec((_BLK, 1), lambda i: (i, 0)),
            pl.BlockSpec((_NC, _BLK, 16), lambda i: (0, i, 0)),
        ],
        out_shape=[
            jax.ShapeDtypeStruct((_N, 1), _f32),
            jax.ShapeDtypeStruct((_NC, _N2, 16), _f32),
        ],
        name="tc_prep",
    )(dpa, dpb, x, W1)


def _tcmid_body(a_ref, hp_ref, dis_ref, b_ref, g_ref, be_ref, m_ref, v_ref,
                w2_ref, out_ref, *, matmul):
    agg = jnp.concatenate([a_ref[0], a_ref[1]], axis=-1)
    hp = jnp.concatenate([hp_ref[0], hp_ref[1]], axis=-1)
    dis = dis_ref[...]
    conv = dis * (agg + hp) + b_ref[...]
    z = jnp.maximum(
        (conv - m_ref[...]) * lax.rsqrt(v_ref[...] + 1e-5) * g_ref[...]
        + be_ref[...], 0.0)
    if matmul:
        h2 = jnp.dot(z, w2_ref[...], preferred_element_type=_f32)
        hh = h2 * dis
        out_ref[0] = hh[:, :16]
        out_ref[1] = hh[:, 16:]
    else:
        out_ref[...] = z


def _tcmid_call(agg, hps, dis2d, b, g, be, m, v, W2, matmul, name):
    import functools
    grid = (_N // _BLK,)
    body = functools.partial(_tcmid_body, matmul=matmul)
    if matmul:
        out_spec = pl.BlockSpec((_NC, _BLK, 16), lambda i: (0, i, 0))
        out_shape = jax.ShapeDtypeStruct((_NC, _N2, 16), _f32)
    else:
        out_spec = pl.BlockSpec((_BLK, _H), lambda i: (i, 0))
        out_shape = jax.ShapeDtypeStruct((_NP, _H), _f32)
    return pl.pallas_call(
        body,
        grid=grid,
        in_specs=[
            pl.BlockSpec((_NC, _BLK, 16), lambda i: (0, i, 0)),
            pl.BlockSpec((_NC, _BLK, 16), lambda i: (0, i, 0)),
            pl.BlockSpec((_BLK, 1), lambda i: (i, 0)),
            pl.BlockSpec((1, _H), lambda i: (0, 0)),
            pl.BlockSpec((1, _H), lambda i: (0, 0)),
            pl.BlockSpec((1, _H), lambda i: (0, 0)),
            pl.BlockSpec((1, _H), lambda i: (0, 0)),
            pl.BlockSpec((1, _H), lambda i: (0, 0)),
            pl.BlockSpec((_H, _H), lambda i: (0, 0)),
        ],
        out_specs=out_spec,
        out_shape=out_shape,
        name=name,
    )(agg, hps, dis2d, b, g, be, m, v, W2)


def _tch_body(sums_ref, cnts_ref, gf_ref,
              ws1_ref, bs1_ref, gs1_ref, bes1_ref, ms1_ref, vs1_ref,
              ws2_ref, bs2_ref, gs2_ref, bes2_ref, ms2_ref, vs2_ref,
              wo_ref, bo_ref, wb_ref, bb_ref, or_ref, bl_ref):
    sums = sums_ref[0] + sums_ref[1]
    cnts = cnts_ref[0] + cnts_ref[1]
    ge = sums / jnp.clip(cnts, 1.0)
    comb = jnp.concatenate([ge, gf_ref[...]], axis=1)
    s = jnp.dot(comb, ws1_ref[...], preferred_element_type=_f32) + bs1_ref[...]
    s = jnp.maximum(
        (s - ms1_ref[...]) * lax.rsqrt(vs1_ref[...] + 1e-5) * gs1_ref[...]
        + bes1_ref[...], 0.0)
    s = jnp.dot(s, ws2_ref[...], preferred_element_type=_f32) + bs2_ref[...]
    s = jnp.maximum(
        (s - ms2_ref[...]) * lax.rsqrt(vs2_ref[...] + 1e-5) * gs2_ref[...]
        + bes2_ref[...], 0.0)
    or_ref[...] = jax.nn.sigmoid(
        jnp.dot(s, wo_ref[...], preferred_element_type=_f32) + bo_ref[...])
    bl_ref[...] = jax.nn.sigmoid(
        jnp.dot(s, wb_ref[...], preferred_element_type=_f32) + bb_ref[...])


def _tch_call(sums, cnts3, gf, Ws1, bs1, gs1, bes1, ms1, vs1,
              Ws2, bs2, gs2, bes2, ms2, vs2, Wo, bo, Wb, bb):
    return pl.pallas_call(
        _tch_body,
        out_shape=[
            jax.ShapeDtypeStruct((_B, 1), _f32),
            jax.ShapeDtypeStruct((_B, 1), _f32),
        ],
        name="tc_head",
    )(sums, cnts3, gf, Ws1, bs1, gs1, bes1, ms1, vs1,
      Ws2, bs2, gs2, bes2, ms2, vs2, Wo, bo, Wb, bb)


def kernel(x, edge_index, edge_weight, batch, global_features,
           W1, b1, g1, be1, m1, v1,
           W2, b2, g2, be2, m2, v2,
           Ws1, bs1, gs1, bes1, ms1, vs1,
           Ws2, bs2, gs2, bes2, ms2, vs2,
           Wo, bo, Wb, bb):
    i32 = jnp.int32
    ei3 = edge_index.reshape(2, _ER, 128)
    w3 = edge_weight.reshape(_ER, 128)
    r2, c2, w2 = _pad_call(ei3, w3)

    degp = _deg_call(c2, w2)
    dpa = degp[:_N].reshape(_N, 1)
    dpb = degp[_N2:_N2 + _N].reshape(_N, 1)

    dis2d, hp1s = _tcb_call(dpa, dpb, x, W1)
    agg1 = _conv_call(r2, c2, w2, hp1s)

    hp2s = _tcmid_call(agg1, hp1s, dis2d,
                       b1.reshape(1, _H), g1.reshape(1, _H),
                       be1.reshape(1, _H), m1.reshape(1, _H),
                       v1.reshape(1, _H), W2, matmul=True, name="tc_mid1")
    agg2 = _conv_call(r2, c2, w2, hp2s)

    z2 = _tcmid_call(agg2, hp2s, dis2d,
                     b2.reshape(1, _H), g2.reshape(1, _H), be2.reshape(1, _H),
                     m2.reshape(1, _H), v2.reshape(1, _H), W2,
                     matmul=False, name="tc_mid2")

    bp = jnp.concatenate([batch, jnp.full((_NP - _N,), _B2S - 1, i32)]
                         ).reshape(_NPG, 128)
    sums_o, cnts_o = _pool_call(z2, bp)
    sums = sums_o.reshape(_NC, _B2S, _H)[:, :_B]
    cnts3 = cnts_o.reshape(_NC, _B2S)[:, :_B].reshape(_NC, _B, 1)

    orange, blue = _tch_call(
        sums, cnts3, global_features,
        Ws1, bs1.reshape(1, 128), gs1.reshape(1, 128), bes1.reshape(1, 128),
        ms1.reshape(1, 128), vs1.reshape(1, 128),
        Ws2, bs2.reshape(1, 64), gs2.reshape(1, 64), bes2.reshape(1, 64),
        ms2.reshape(1, 64), vs2.reshape(1, 64),
        Wo, bo.reshape(1, 1), Wb, bb.reshape(1, 1))
    return (orange, blue)


# speedup vs baseline: 42.7562x; 1.1003x over previous
import jax
import jax.numpy as jnp
from jax import lax
from jax.experimental import pallas as pl
from jax.experimental.pallas import tpu as pltpu
from jax.experimental.pallas import tpu_sc as plsc

_N = 100000
_E = 3200000
_B = 10000
_H = 32
_NC = 2
_NS = 16
_EP_ROWS = 25088
_EP = _EP_ROWS * 128
_N2 = 100096
_B2 = 10112
_NP = 102400
_NPG = _NP // 128

_mesh = plsc.VectorSubcoreMesh(core_axis_name="c", subcore_axis_name="s")
_f32 = jnp.float32
_SC_PARAMS = pltpu.CompilerParams(use_tc_tiling_on_sc=False)


def _deg_body(c_hbm, w_hbm, out_hbm, acc, zbuf, cst0, wst0, cst1, wst1,
              sem_i0, sem_s0, sem_i1, sem_s1):
    core = lax.axis_index("c")
    sub = lax.axis_index("s")
    wid = core * _NS + sub
    n_sl = _N2 // _NS
    zv = jnp.zeros((16,), _f32)

    @pl.loop(0, 64)
    def _(i):
        zbuf[pl.ds(i * 16, 16)] = zv

    @pl.loop(0, 6)
    def _(k):
        pltpu.sync_copy(zbuf, acc.at[pl.ds(sub * n_sl + k * 1024, 1024)])

    pltpu.sync_copy(zbuf.at[pl.ds(0, 112)],
                    acc.at[pl.ds(sub * n_sl + 6144, 112)])
    plsc.subcore_barrier()
    rows_per_tile = _EP_ROWS // (_NC * _NS)
    ngroups = rows_per_tile // 8
    row_base = wid * rows_per_tile

    def fire_idx(g, cst, wst, sem):
        row0 = row_base + g * 8
        pltpu.async_copy(c_hbm.at[pl.ds(row0, 8)], cst, sem)
        pltpu.async_copy(w_hbm.at[pl.ds(row0, 8)], wst, sem)

    def wait_idx(cst, wst, sem):
        pltpu.make_async_copy(c_hbm.at[pl.ds(0, 8)], cst, sem).wait()
        pltpu.make_async_copy(w_hbm.at[pl.ds(0, 8)], wst, sem).wait()

    def wait_scatters(wst, sem):
        for j in range(8):
            pltpu.make_async_copy(wst.at[j], acc.at[pl.ds(0, 128)],
                                  sem).wait()

    def process(g, cstP, wstP, sem_iP, sem_sP, cstQ, wstQ, sem_iQ, sem_sQ):
        @pl.when(g > 0)
        def _():
            wait_scatters(wstQ, sem_sQ)

        @pl.when(g + 1 < ngroups)
        def _():
            fire_idx(g + 1, cstQ, wstQ, sem_iQ)

        wait_idx(cstP, wstP, sem_iP)
        for j in range(8):
            pltpu.async_copy(wstP.at[j], acc.at[cstP.at[j]], sem_sP,
                             add=True)

    fire_idx(0, cst0, wst0, sem_i0)

    @pl.loop(0, ngroups // 2)
    def _(k):
        g = k * 2
        process(g, cst0, wst0, sem_i0, sem_s0, cst1, wst1, sem_i1, sem_s1)
        process(g + 1, cst1, wst1, sem_i1, sem_s1, cst0, wst0, sem_i0,
                sem_s0)

    wait_scatters(wst1, sem_s1)
    plsc.subcore_barrier()
    base = core * _N2 + sub * n_sl

    @pl.loop(0, 6)
    def _(k):
        pltpu.sync_copy(acc.at[pl.ds(sub * n_sl + k * 1024, 1024)], zbuf)
        pltpu.sync_copy(zbuf, out_hbm.at[pl.ds(base + k * 1024, 1024)])

    pltpu.sync_copy(acc.at[pl.ds(sub * n_sl + 6144, 112)],
                    zbuf.at[pl.ds(0, 112)])
    pltpu.sync_copy(zbuf.at[pl.ds(0, 112)],
                    out_hbm.at[pl.ds(base + 6144, 112)])


def _deg_call(c2, w2):
    return pl.kernel(
        _deg_body,
        out_type=jax.ShapeDtypeStruct((_NC * _N2,), _f32),
        mesh=_mesh,
        scratch_types=[
            pltpu.VMEM_SHARED((_N2,), _f32),
            pltpu.VMEM((1024,), _f32),
            pltpu.VMEM((8, 128), jnp.int32),
            pltpu.VMEM((8, 128), _f32),
            pltpu.VMEM((8, 128), jnp.int32),
            pltpu.VMEM((8, 128), _f32),
            pltpu.SemaphoreType.DMA,
            pltpu.SemaphoreType.DMA,
            pltpu.SemaphoreType.DMA,
            pltpu.SemaphoreType.DMA,
        ],
        name="sc_deg",
        compiler_params=_SC_PARAMS,
    )(c2, w2)


_SROWS = 4
_SEDGES = _SROWS * 128
_NSTAGES = (_EP_ROWS // _NS) // _SROWS


def _conv_body(r_hbm, c_hbm, w_hbm, hp_hbm, out_hbm,
               acc,
               rst0, wst0, rows0, rst1, wst1, rows1,
               cstA, cstB, cstC, cstD,
               sem_i0, sem_g0, sem_s0, sem_i1, sem_g1, sem_s1):
    core = lax.axis_index("c")
    sub = lax.axis_index("s")
    n_sl = _N2 // _NS
    zv = jnp.zeros((1, 16), _f32)

    @pl.loop(0, _SEDGES)
    def _(i):
        rows0[pl.ds(i, 1), :] = zv

    @pl.loop(0, 12)
    def _(k):
        pltpu.sync_copy(rows0, acc.at[pl.ds(sub * n_sl + k * 512, 512)])

    pltpu.sync_copy(rows0.at[pl.ds(0, 112)],
                    acc.at[pl.ds(sub * n_sl + 6144, 112)])
    plsc.subcore_barrier()
    row_base = sub * (_EP_ROWS // _NS)

    def fire_idx(s, rst, cst, wst, sem):
        row0 = row_base + s * _SROWS
        pltpu.async_copy(r_hbm.at[pl.ds(row0, _SROWS)], rst, sem)
        pltpu.async_copy(c_hbm.at[pl.ds(row0, _SROWS)], cst, sem)
        pltpu.async_copy(w_hbm.at[pl.ds(row0, _SROWS)], wst, sem)

    def wait_idx(rst, cst, wst, sem):
        pltpu.make_async_copy(r_hbm.at[pl.ds(0, _SROWS)], rst, sem).wait()
        pltpu.make_async_copy(c_hbm.at[pl.ds(0, _SROWS)], cst, sem).wait()
        pltpu.make_async_copy(w_hbm.at[pl.ds(0, _SROWS)], wst, sem).wait()

    def fire_gathers(rst, rows, sem):
        for j in range(_SROWS):
            pltpu.async_copy(hp_hbm.at[core].at[rst.at[j]],
                             rows.at[pl.ds(j * 128, 128), :], sem)

    def wait_gathers(rows, sem):
        pltpu.make_async_copy(hp_hbm.at[core, pl.ds(0, _SEDGES), :], rows,
                              sem).wait()

    def wait_scatters(rows, sem):
        pltpu.make_async_copy(rows, acc.at[pl.ds(0, _SEDGES)], sem).wait()

    def scale_and_scatter(cst, wst, rows, sem):
        for j in range(_SROWS):
            @pl.loop(0, 8)
            def _(q):
                w16 = wst.at[j][pl.ds(q * 16, 16)]
                for e in range(16):
                    idx = j * 128 + q * 16 + e
                    rows[pl.ds(idx, 1), :] = rows[pl.ds(idx, 1), :] * w16[e]

            pltpu.async_copy(rows.at[pl.ds(j * 128, 128), :],
                             acc.at[cst.at[j]], sem, add=True)

    fire_idx(0, rst0, cstA, wst0, sem_i0)
    wait_idx(rst0, cstA, wst0, sem_i0)
    fire_gathers(rst0, rows0, sem_g0)
    fire_idx(1, rst1, cstB, wst1, sem_i1)

    def process(s, rstP, wstP, cstS, rowsP, sem_iP, sem_gP, sem_sP,
                rstQ, wstQ, cstQn, rowsQ, sem_iQ, sem_gQ, sem_sQ, cstNext):
        @pl.when(s > 0)
        def _():
            wait_scatters(rowsQ, sem_sQ)

        @pl.when(s + 1 < _NSTAGES)
        def _():
            wait_idx(rstQ, cstQn, wstQ, sem_iQ)
            fire_gathers(rstQ, rowsQ, sem_gQ)

        wait_gathers(rowsP, sem_gP)
        scale_and_scatter(cstS, wstP, rowsP, sem_sP)

        @pl.when(s + 2 < _NSTAGES)
        def _():
            fire_idx(s + 2, rstP, cstNext, wstP, sem_iP)

    @pl.loop(0, _NSTAGES // 4)
    def _(m):
        s = m * 4
        process(s, rst0, wst0, cstA, rows0, sem_i0, sem_g0, sem_s0,
                rst1, wst1, cstB, rows1, sem_i1, sem_g1, sem_s1, cstC)
        process(s + 1, rst1, wst1, cstB, rows1, sem_i1, sem_g1, sem_s1,
                rst0, wst0, cstC, rows0, sem_i0, sem_g0, sem_s0, cstD)
        process(s + 2, rst0, wst0, cstC, rows0, sem_i0, sem_g0, sem_s0,
                rst1, wst1, cstD, rows1, sem_i1, sem_g1, sem_s1, cstA)
        process(s + 3, rst1, wst1, cstD, rows1, sem_i1, sem_g1, sem_s1,
                rst0, wst0, cstA, rows0, sem_i0, sem_g0, sem_s0, cstB)

    wait_scatters(rows1, sem_s1)
    plsc.subcore_barrier()

    @pl.loop(0, 12)
    def _(k):
        pltpu.sync_copy(acc.at[pl.ds(sub * n_sl + k * 512, 512)], rows0)
        pltpu.sync_copy(rows0, out_hbm.at[core, pl.ds(sub * n_sl + k * 512,
                                                      512)])

    pltpu.sync_copy(acc.at[pl.ds(sub * n_sl + 6144, 112)],
                    rows0.at[pl.ds(0, 112)])
    pltpu.sync_copy(rows0.at[pl.ds(0, 112)],
                    out_hbm.at[core, pl.ds(sub * n_sl + 6144, 112)])


def _conv_call(r2, c2, w2, hps):
    return pl.kernel(
        _conv_body,
        out_type=jax.ShapeDtypeStruct((_NC, _N2, 16), _f32),
        mesh=_mesh,
        scratch_types=[
            pltpu.VMEM_SHARED((_N2, 16), _f32),
            pltpu.VMEM((_SROWS, 128), jnp.int32),
            pltpu.VMEM((_SROWS, 128), _f32),
            pltpu.VMEM((_SEDGES, 16), _f32),
            pltpu.VMEM((_SROWS, 128), jnp.int32),
            pltpu.VMEM((_SROWS, 128), _f32),
            pltpu.VMEM((_SEDGES, 16), _f32),
            pltpu.VMEM((_SROWS, 128), jnp.int32),
            pltpu.VMEM((_SROWS, 128), jnp.int32),
            pltpu.VMEM((_SROWS, 128), jnp.int32),
            pltpu.VMEM((_SROWS, 128), jnp.int32),
            pltpu.SemaphoreType.DMA,
            pltpu.SemaphoreType.DMA,
            pltpu.SemaphoreType.DMA,
            pltpu.SemaphoreType.DMA,
            pltpu.SemaphoreType.DMA,
            pltpu.SemaphoreType.DMA,
        ],
        name="sc_conv",
        compiler_params=_SC_PARAMS,
    )(r2, c2, w2, hps)


_B2S = 10240


def _pool_body(z_hbm, b_hbm, sums_hbm, cnts_hbm,
               accS, accC, zbuf, zbufc, ones, zstage, bstage):
    core = lax.axis_index("c")
    sub = lax.axis_index("s")
    wid = core * _NS + sub
    b_sl = _B2S // _NS
    zv = jnp.zeros((16,), _f32)
    ov = jnp.ones((1, 16), _f32)

    @pl.loop(0, 640)
    def _(i):
        zbuf[pl.ds(i, 1), :] = jnp.zeros((1, _H), _f32)

    @pl.loop(0, 40)
    def _(i):
        zbufc[pl.ds(i * 16, 16)] = zv

    for k in range(8):
        ones[pl.ds(0, 1), pl.ds(k * 16, 16)] = ov

    pltpu.sync_copy(zbuf, accS.at[pl.ds(sub * b_sl, b_sl)])
    pltpu.sync_copy(zbufc, accC.at[pl.ds(sub * b_sl, b_sl)])
    plsc.subcore_barrier()
    groups_per_tile = _NPG // (_NC * _NS)

    @pl.loop(0, groups_per_tile)
    def _(g):
        grp = wid * groups_per_tile + g
        pltpu.sync_copy(z_hbm.at[pl.ds(grp * 128, 128)], zstage)
        pltpu.sync_copy(b_hbm.at[pl.ds(grp, 1)], bstage)
        pltpu.sync_copy(zstage, accS.at[bstage.at[0]], add=True)
        pltpu.sync_copy(ones.at[0], accC.at[bstage.at[0]], add=True)

    plsc.subcore_barrier()
    pltpu.sync_copy(accS.at[pl.ds(sub * b_sl, b_sl)], zbuf)
    pltpu.sync_copy(zbuf, sums_hbm.at[pl.ds(core * _B2S + sub * b_sl, b_sl)])
    pltpu.sync_copy(accC.at[pl.ds(sub * b_sl, b_sl)], zbufc)
    pltpu.sync_copy(zbufc,
                    cnts_hbm.at[pl.ds(core * _B2S + sub * b_sl, b_sl)])


def _pool_call(z2p, bp):
    return pl.kernel(
        _pool_body,
        out_type=(jax.ShapeDtypeStruct((_NC * _B2S, _H), _f32),
                  jax.ShapeDtypeStruct((_NC * _B2S,), _f32)),
        mesh=_mesh,
        scratch_types=[
            pltpu.VMEM_SHARED((_B2S, _H), _f32),
            pltpu.VMEM_SHARED((_B2S,), _f32),
            pltpu.VMEM((640, _H), _f32),
            pltpu.VMEM((640,), _f32),
            pltpu.VMEM((1, 128), _f32),
            pltpu.VMEM((128, _H), _f32),
            pltpu.VMEM((1, 128), jnp.int32),
        ],
        name="sc_pool",
        compiler_params=_SC_PARAMS,
    )(z2p, bp)


_BLK = 5000


_ER = _E // 128
_PBLK = 1792


def _pad_body(ei_ref, w_ref, r_ref, c_ref, w2_ref):
    i = pl.program_id(0)
    rid = lax.broadcasted_iota(jnp.int32, (_PBLK, 128), 0) + i * _PBLK
    m = rid < _ER
    r_ref[...] = jnp.where(m, ei_ref[0], 0)
    c_ref[...] = jnp.where(m, ei_ref[1], 0)
    w2_ref[...] = jnp.where(m, w_ref[...], 0.0)


def _pad_call(ei3, w3):
    return pl.pallas_call(
        _pad_body,
        grid=(_EP_ROWS // _PBLK,),
        in_specs=[
            pl.BlockSpec((2, _PBLK, 128), lambda i: (0, i, 0)),
            pl.BlockSpec((_PBLK, 128), lambda i: (i, 0)),
        ],
        out_specs=[
            pl.BlockSpec((_PBLK, 128), lambda i: (i, 0)),
            pl.BlockSpec((_PBLK, 128), lambda i: (i, 0)),
            pl.BlockSpec((_PBLK, 128), lambda i: (i, 0)),
        ],
        out_shape=[
            jax.ShapeDtypeStruct((_EP_ROWS, 128), jnp.int32),
            jax.ShapeDtypeStruct((_EP_ROWS, 128), jnp.int32),
            jax.ShapeDtypeStruct((_EP_ROWS, 128), _f32),
        ],
        name="tc_pad",
    )(ei3, w3)


def _tcb_body(dpa_ref, dpb_ref, x_ref, w1_ref, dis_ref, hp_ref):
    deg = 1.0 + dpa_ref[:, 0] + dpb_ref[:, 0]
    dis = lax.rsqrt(deg)
    h = jnp.dot(x_ref[...], w1_ref[...], preferred_element_type=_f32)
    hp = h * dis[:, None]
    hp_ref[0] = hp[:, :16]
    hp_ref[1] = hp[:, 16:]
    dis_ref[...] = dis[:, None]


def _tcb_call(dpa, dpb, x, W1):
    grid = (_N // _BLK,)
    return pl.pallas_call(
        _tcb_body,
        grid=grid,
        in_specs=[
            pl.BlockSpec((_BLK, 1), lambda i: (i, 0)),
            pl.BlockSpec((_BLK, 1), lambda i: (i, 0)),
            pl.BlockSpec((_BLK, 13), lambda i: (i, 0)),
            pl.BlockSpec((13, _H), lambda i: (0, 0)),
        ],
        out_specs=[
            pl.BlockSpec((_BLK, 1), lambda i: (i, 0)),
            pl.BlockSpec((_NC, _BLK, 16), lambda i: (0, i, 0)),
        ],
        out_shape=[
            jax.ShapeDtypeStruct((_N, 1), _f32),
            jax.ShapeDtypeStruct((_NC, _N2, 16), _f32),
        ],
        name="tc_prep",
    )(dpa, dpb, x, W1)


def _tcmid_body(a_ref, hp_ref, dis_ref, b_ref, g_ref, be_ref, m_ref, v_ref,
                w2_ref, out_ref, *, matmul):
    agg = jnp.concatenate([a_ref[0], a_ref[1]], axis=-1)
    hp = jnp.concatenate([hp_ref[0], hp_ref[1]], axis=-1)
    dis = dis_ref[...]
    conv = dis * (agg + hp) + b_ref[...]
    z = jnp.maximum(
        (conv - m_ref[...]) * lax.rsqrt(v_ref[...] + 1e-5) * g_ref[...]
        + be_ref[...], 0.0)
    if matmul:
        h2 = jnp.dot(z, w2_ref[...], preferred_element_type=_f32)
        hh = h2 * dis
        out_ref[0] = hh[:, :16]
        out_ref[1] = hh[:, 16:]
    else:
        out_ref[...] = z


def _tcmid_call(agg, hps, dis2d, b, g, be, m, v, W2, matmul, name):
    import functools
    grid = (_N // _BLK,)
    body = functools.partial(_tcmid_body, matmul=matmul)
    if matmul:
        out_spec = pl.BlockSpec((_NC, _BLK, 16), lambda i: (0, i, 0))
        out_shape = jax.ShapeDtypeStruct((_NC, _N2, 16), _f32)
    else:
        out_spec = pl.BlockSpec((_BLK, _H), lambda i: (i, 0))
        out_shape = jax.ShapeDtypeStruct((_NP, _H), _f32)
    return pl.pallas_call(
        body,
        grid=grid,
        in_specs=[
            pl.BlockSpec((_NC, _BLK, 16), lambda i: (0, i, 0)),
            pl.BlockSpec((_NC, _BLK, 16), lambda i: (0, i, 0)),
            pl.BlockSpec((_BLK, 1), lambda i: (i, 0)),
            pl.BlockSpec((1, _H), lambda i: (0, 0)),
            pl.BlockSpec((1, _H), lambda i: (0, 0)),
            pl.BlockSpec((1, _H), lambda i: (0, 0)),
            pl.BlockSpec((1, _H), lambda i: (0, 0)),
            pl.BlockSpec((1, _H), lambda i: (0, 0)),
            pl.BlockSpec((_H, _H), lambda i: (0, 0)),
        ],
        out_specs=out_spec,
        out_shape=out_shape,
        name=name,
    )(agg, hps, dis2d, b, g, be, m, v, W2)


def _tch_body(sums_ref, cnts_ref, gf_ref,
              ws1_ref, bs1_ref, gs1_ref, bes1_ref, ms1_ref, vs1_ref,
              ws2_ref, bs2_ref, gs2_ref, bes2_ref, ms2_ref, vs2_ref,
              wo_ref, bo_ref, wb_ref, bb_ref, or_ref, bl_ref):
    sums = sums_ref[0] + sums_ref[1]
    cnts = cnts_ref[0] + cnts_ref[1]
    ge = sums / jnp.clip(cnts, 1.0)
    comb = jnp.concatenate([ge, gf_ref[...]], axis=1)
    s = jnp.dot(comb, ws1_ref[...], preferred_element_type=_f32) + bs1_ref[...]
    s = jnp.maximum(
        (s - ms1_ref[...]) * lax.rsqrt(vs1_ref[...] + 1e-5) * gs1_ref[...]
        + bes1_ref[...], 0.0)
    s = jnp.dot(s, ws2_ref[...], preferred_element_type=_f32) + bs2_ref[...]
    s = jnp.maximum(
        (s - ms2_ref[...]) * lax.rsqrt(vs2_ref[...] + 1e-5) * gs2_ref[...]
        + bes2_ref[...], 0.0)
    or_ref[...] = jax.nn.sigmoid(
        jnp.dot(s, wo_ref[...], preferred_element_type=_f32) + bo_ref[...])
    bl_ref[...] = jax.nn.sigmoid(
        jnp.dot(s, wb_ref[...], preferred_element_type=_f32) + bb_ref[...])


def _tch_call(sums, cnts3, gf, Ws1, bs1, gs1, bes1, ms1, vs1,
              Ws2, bs2, gs2, bes2, ms2, vs2, Wo, bo, Wb, bb):
    return pl.pallas_call(
        _tch_body,
        out_shape=[
            jax.ShapeDtypeStruct((_B, 1), _f32),
            jax.ShapeDtypeStruct((_B, 1), _f32),
        ],
        name="tc_head",
    )(sums, cnts3, gf, Ws1, bs1, gs1, bes1, ms1, vs1,
      Ws2, bs2, gs2, bes2, ms2, vs2, Wo, bo, Wb, bb)


def kernel(x, edge_index, edge_weight, batch, global_features,
           W1, b1, g1, be1, m1, v1,
           W2, b2, g2, be2, m2, v2,
           Ws1, bs1, gs1, bes1, ms1, vs1,
           Ws2, bs2, gs2, bes2, ms2, vs2,
           Wo, bo, Wb, bb):
    i32 = jnp.int32
    ei3 = edge_index.reshape(2, _ER, 128)
    w3 = edge_weight.reshape(_ER, 128)
    r2, c2, w2 = _pad_call(ei3, w3)

    degp = _deg_call(c2, w2)
    dpa = degp[:_N].reshape(_N, 1)
    dpb = degp[_N2:_N2 + _N].reshape(_N, 1)

    dis2d, hp1s = _tcb_call(dpa, dpb, x, W1)
    agg1 = _conv_call(r2, c2, w2, hp1s)

    hp2s = _tcmid_call(agg1, hp1s, dis2d,
                       b1.reshape(1, _H), g1.reshape(1, _H),
                       be1.reshape(1, _H), m1.reshape(1, _H),
                       v1.reshape(1, _H), W2, matmul=True, name="tc_mid1")
    agg2 = _conv_call(r2, c2, w2, hp2s)

    z2 = _tcmid_call(agg2, hp2s, dis2d,
                     b2.reshape(1, _H), g2.reshape(1, _H), be2.reshape(1, _H),
                     m2.reshape(1, _H), v2.reshape(1, _H), W2,
                     matmul=False, name="tc_mid2")

    bp = jnp.concatenate([batch, jnp.full((_NP - _N,), _B2S - 1, i32)]
                         ).reshape(_NPG, 128)
    sums_o, cnts_o = _pool_call(z2, bp)
    sums = sums_o.reshape(_NC, _B2S, _H)[:, :_B]
    cnts3 = cnts_o.reshape(_NC, _B2S)[:, :_B].reshape(_NC, _B, 1)

    orange, blue = _tch_call(
        sums, cnts3, global_features,
        Ws1, bs1.reshape(1, 128), gs1.reshape(1, 128), bes1.reshape(1, 128),
        ms1.reshape(1, 128), vs1.reshape(1, 128),
        Ws2, bs2.reshape(1, 64), gs2.reshape(1, 64), bes2.reshape(1, 64),
        ms2.reshape(1, 64), vs2.reshape(1, 64),
        Wo, bo.reshape(1, 1), Wb, bb.reshape(1, 1))
    return (orange, blue)
